# Initial kernel scaffold; baseline (speedup 1.0000x reference)
import jax
import jax.numpy as jnp
from jax.experimental import pallas as pl


def kernel(X, edge_index, edge_weight, W1, b1, W2, b2, W3, b3):
    raise NotImplementedError("write your pallas kernel here")


# speedup vs baseline: 6.7488x; 6.7488x over previous
import functools

import jax
import jax.numpy as jnp
from jax import lax
from jax.experimental import pallas as pl
from jax.experimental.pallas import tpu as pltpu
from jax.experimental.pallas import tpu_sc as plsc

N_PAD = 10240
D = 128
NW = 32
K = 128
IDXB = 16
ROWS_PER_TILE = N_PAD // 16
TC_BLK = 256

_MESH = dict(core_axis_name="c", subcore_axis_name="s", num_cores=2,
             num_subcores=16)


def _sc_deg_body(nchunks, col_hbm, ew_hbm, out_hbm, colb, ewb, zv, deg_sh):
    c = lax.axis_index("c")
    s = lax.axis_index("s")
    wid = s * 2 + c

    zero16 = jnp.zeros((16,), jnp.float32)

    def zbody(i, _):
        zv[pl.ds(i * 16, 16)] = zero16
        return 0

    lax.fori_loop(0, ROWS_PER_TILE // 16, zbody, 0, unroll=8)
    pltpu.sync_copy(zv, deg_sh.at[pl.ds(s * ROWS_PER_TILE, ROWS_PER_TILE)])
    plsc.subcore_barrier()

    def block(bi, _):
        pltpu.sync_copy(col_hbm.at[wid, pl.ds(bi * IDXB, IDXB)], colb)
        pltpu.sync_copy(ew_hbm.at[wid, pl.ds(bi * IDXB * K, IDXB * K)], ewb)

        def chunk(jj, _):
            pltpu.sync_copy(ewb.at[pl.ds(jj * K, K)], deg_sh.at[colb.at[jj]],
                            add=True)
            return 0

        lax.fori_loop(0, IDXB, chunk, 0)
        return 0

    lax.fori_loop(0, nchunks // IDXB, block, 0)
    plsc.subcore_barrier()
    pltpu.sync_copy(deg_sh.at[pl.ds(s * ROWS_PER_TILE, ROWS_PER_TILE)],
                    out_hbm.at[c, pl.ds(s * ROWS_PER_TILE, ROWS_PER_TILE)])


def _sc_deg(col3, ew2):
    nchunks = col3.shape[1]
    kern = pl.kernel(
        functools.partial(_sc_deg_body, nchunks),
        out_type=jax.ShapeDtypeStruct((2, N_PAD), jnp.float32),
        mesh=plsc.VectorSubcoreMesh(**_MESH),
        compiler_params=pltpu.CompilerParams(needs_layout_passes=False),
        scratch_types=[
            pltpu.VMEM((IDXB, K), jnp.int32),
            pltpu.VMEM((IDXB * K,), jnp.float32),
            pltpu.VMEM((ROWS_PER_TILE,), jnp.float32),
            pltpu.VMEM_SHARED((N_PAD,), jnp.float32),
        ],
    )
    return kern(col3, ew2)


def _sc_layer_body(nchunks, h_hbm, row_hbm, col_hbm, ew_hbm, out_hbm,
                   rowb, colb, ewb, gbuf, sbuf, acc_sh):
    c = lax.axis_index("c")
    s = lax.axis_index("s")
    wid = s * 2 + c

    zero16 = jnp.zeros((16,), jnp.float32)

    def zbody(r, _):
        for i in range(8):
            gbuf[r, pl.ds(16 * i, 16)] = zero16
        return 0

    lax.fori_loop(0, K, zbody, 0)
    for k in range(ROWS_PER_TILE // K):
        pltpu.sync_copy(gbuf, acc_sh.at[pl.ds(s * ROWS_PER_TILE + k * K, K)])
    plsc.subcore_barrier()

    def block(bi, _):
        pltpu.sync_copy(row_hbm.at[wid, pl.ds(bi * IDXB, IDXB)], rowb)
        pltpu.sync_copy(col_hbm.at[wid, pl.ds(bi * IDXB, IDXB)], colb)
        pltpu.sync_copy(ew_hbm.at[wid, pl.ds(bi * IDXB * K, IDXB * K)], ewb)

        def chunk(jj, _):
            pltpu.sync_copy(h_hbm.at[rowb.at[jj]], gbuf)

            def edge(e, _):
                ewbc = plsc.load_gather(
                    ewb, [jnp.full((16,), jj * K + e, jnp.int32)])
                for i in range(8):
                    sbuf[e, pl.ds(16 * i, 16)] = gbuf[e, pl.ds(16 * i, 16)] * ewbc
                return 0

            lax.fori_loop(0, K, edge, 0, unroll=2)
            pltpu.sync_copy(sbuf, acc_sh.at[colb.at[jj]], add=True)
            return 0

        lax.fori_loop(0, IDXB, chunk, 0)
        return 0

    lax.fori_loop(0, nchunks // IDXB, block, 0)
    plsc.subcore_barrier()
    pltpu.sync_copy(acc_sh.at[pl.ds(s * ROWS_PER_TILE, ROWS_PER_TILE)],
                    out_hbm.at[c, pl.ds(s * ROWS_PER_TILE, ROWS_PER_TILE)])


def _sc_layer(h, row3, col3, ew2):
    nchunks = row3.shape[1]
    kern = pl.kernel(
        functools.partial(_sc_layer_body, nchunks),
        out_type=jax.ShapeDtypeStruct((2, N_PAD, D), jnp.float32),
        mesh=plsc.VectorSubcoreMesh(**_MESH),
        compiler_params=pltpu.CompilerParams(needs_layout_passes=False),
        scratch_types=[
            pltpu.VMEM((IDXB, K), jnp.int32),
            pltpu.VMEM((IDXB, K), jnp.int32),
            pltpu.VMEM((IDXB * K,), jnp.float32),
            pltpu.VMEM((K, D), jnp.float32),
            pltpu.VMEM((K, D), jnp.float32),
            pltpu.VMEM_SHARED((N_PAD, D), jnp.float32),
        ],
    )
    return kern(h, row3, col3, ew2)


def _dotT(x, w):
    return lax.dot_general(x, w, (((1,), (1,)), ((), ())),
                           preferred_element_type=jnp.float32)


def _tc_first_body(x_ref, w_ref, deg_ref, h_ref, dinv_ref):
    deg = deg_ref[0] + deg_ref[1] + 1.0
    dinv = jnp.where(deg > 0.0, lax.rsqrt(deg), 0.0)
    h_ref[...] = dinv * _dotT(x_ref[...], w_ref[...])
    dinv_ref[...] = dinv


def _tc_first(x, w, deg2):
    grid = (N_PAD // TC_BLK,)
    return pl.pallas_call(
        _tc_first_body,
        grid=grid,
        in_specs=[
            pl.BlockSpec((TC_BLK, D), lambda i: (i, 0)),
            pl.BlockSpec((D, D), lambda i: (0, 0)),
            pl.BlockSpec((2, TC_BLK, 1), lambda i: (0, i, 0)),
        ],
        out_specs=[
            pl.BlockSpec((TC_BLK, D), lambda i: (i, 0)),
            pl.BlockSpec((TC_BLK, 1), lambda i: (i, 0)),
        ],
        out_shape=[
            jax.ShapeDtypeStruct((N_PAD, D), jnp.float32),
            jax.ShapeDtypeStruct((N_PAD, 1), jnp.float32),
        ],
    )(x, w, deg2)


def _tc_mid_body(acc_ref, h_ref, dinv_ref, b_ref, w_ref, out_ref):
    dinv = dinv_ref[...]
    o = dinv * (acc_ref[0] + acc_ref[1] + h_ref[...]) + b_ref[...]
    o = jnp.maximum(o, 0.0)
    out_ref[...] = dinv * _dotT(o, w_ref[...])


def _tc_mid(acc, h, dinv, b, w):
    grid = (N_PAD // TC_BLK,)
    return pl.pallas_call(
        _tc_mid_body,
        grid=grid,
        in_specs=[
            pl.BlockSpec((2, TC_BLK, D), lambda i: (0, i, 0)),
            pl.BlockSpec((TC_BLK, D), lambda i: (i, 0)),
            pl.BlockSpec((TC_BLK, 1), lambda i: (i, 0)),
            pl.BlockSpec((1, D), lambda i: (0, 0)),
            pl.BlockSpec((D, D), lambda i: (0, 0)),
        ],
        out_specs=pl.BlockSpec((TC_BLK, D), lambda i: (i, 0)),
        out_shape=jax.ShapeDtypeStruct((N_PAD, D), jnp.float32),
    )(acc, h, dinv, b, w)


def _tc_last_body(acc_ref, h_ref, dinv_ref, b_ref, out_ref):
    dinv = dinv_ref[...]
    out_ref[...] = dinv * (acc_ref[0] + acc_ref[1] + h_ref[...]) + b_ref[...]


def _tc_last(acc, h, dinv, b):
    grid = (N_PAD // TC_BLK,)
    return pl.pallas_call(
        _tc_last_body,
        grid=grid,
        in_specs=[
            pl.BlockSpec((2, TC_BLK, D), lambda i: (0, i, 0)),
            pl.BlockSpec((TC_BLK, D), lambda i: (i, 0)),
            pl.BlockSpec((TC_BLK, 1), lambda i: (i, 0)),
            pl.BlockSpec((1, D), lambda i: (0, 0)),
        ],
        out_specs=pl.BlockSpec((TC_BLK, D), lambda i: (i, 0)),
        out_shape=jax.ShapeDtypeStruct((N_PAD, D), jnp.float32),
    )(acc, h, dinv, b)


def kernel(X, edge_index, edge_weight, W1, b1, W2, b2, W3, b3):
    n, d = X.shape
    e = edge_weight.shape[0]
    nchunks = -(-e // (NW * K * IDXB)) * IDXB
    e_pad = NW * nchunks * K
    extra = e_pad - e

    xp = jnp.pad(X, ((0, N_PAD - n), (0, 0)))

    ar = jnp.arange(extra, dtype=jnp.int32)
    row = jnp.concatenate([edge_index[0], ar % n])
    col = jnp.concatenate([edge_index[1], n + ar % (N_PAD - n)])
    ew = jnp.concatenate([edge_weight, jnp.zeros((extra,), jnp.float32)])
    row3 = row.reshape(NW, nchunks, K)
    col3 = col.reshape(NW, nchunks, K)
    ew2 = ew.reshape(NW, nchunks * K)

    deg2 = _sc_deg(col3, ew2).reshape(2, N_PAD, 1)

    h1, dinv = _tc_first(xp, W1, deg2)
    acc1 = _sc_layer(h1, row3, col3, ew2)
    h2 = _tc_mid(acc1, h1, dinv, b1.reshape(1, D), W2)
    acc2 = _sc_layer(h2, row3, col3, ew2)
    h3 = _tc_mid(acc2, h2, dinv, b2.reshape(1, D), W3)
    acc3 = _sc_layer(h3, row3, col3, ew2)
    out = _tc_last(acc3, h3, dinv, b3.reshape(1, D))
    return out[:n]
